# baseline (speedup 1.0000x reference)
import jax
import jax.numpy as jnp
from jax.experimental import pallas as pl
from jax.experimental.pallas import tpu as pltpu

_LANE = 128
_TILE_B = 8
_CHUNK = 2048


def _make_gp_kernel(n_valid_rows, inv_b):
    def _gp_kernel(x_ref, out_ref):
        i = pl.program_id(0)

        @pl.when(i == 0)
        def _init():
            out_ref[...] = jnp.zeros_like(out_ref)

        f = x_ref.shape[1]
        acc = jnp.zeros((x_ref.shape[0], _CHUNK), jnp.float32)
        for j in range(0, f, _CHUNK):
            blk = x_ref[:, j : j + _CHUNK].astype(jnp.float32)
            acc = acc + blk * blk
        ssq = jnp.sum(acc, axis=-1, keepdims=True)
        pen = (jnp.sqrt(ssq) - 1.0) ** 2
        if n_valid_rows < pen.shape[0]:
            row = jax.lax.broadcasted_iota(jnp.int32, pen.shape, 0)
            last = pl.num_programs(0) - 1
            pen = jnp.where((i != last) | (row < n_valid_rows), pen, 0.0)
        out_ref[...] += jnp.sum(pen) * inv_b

    return _gp_kernel


def _gradient_penalty(x):
    B, F = x.shape
    B_pad = -(-B // _TILE_B) * _TILE_B
    F_pad = -(-F // _CHUNK) * _CHUNK
    if (B_pad, F_pad) != (B, F):
        x = jnp.pad(x, ((0, B_pad - B), (0, F_pad - F)))

    n_valid_last = _TILE_B - (B_pad - B)
    loss = pl.pallas_call(
        _make_gp_kernel(n_valid_last, 1.0 / B),
        out_shape=jax.ShapeDtypeStruct((1, 1), jnp.float32),
        grid=(B_pad // _TILE_B,),
        in_specs=[pl.BlockSpec((_TILE_B, F_pad), lambda i: (i, 0))],
        out_specs=pl.BlockSpec((1, 1), lambda i: (0, 0)),
        compiler_params=pltpu.CompilerParams(
            dimension_semantics=("arbitrary",),
            vmem_limit_bytes=64 * 1024 * 1024,
        ),
    )(x)

    return loss[0, 0]


def kernel(dydx_flat):
    return _gradient_penalty(dydx_flat)


# speedup vs baseline: 1.1388x; 1.1388x over previous
import jax
import jax.numpy as jnp
from jax.experimental import pallas as pl
from jax.experimental.pallas import tpu as pltpu

_TILE_B = 16
_CHUNK = 2048


def _make_gp_kernel(n_valid_rows, inv_b):
    def _gp_kernel(x_ref, out_ref):
        i = pl.program_id(0)

        @pl.when(i == 0)
        def _init():
            out_ref[...] = jnp.zeros_like(out_ref)

        f = x_ref.shape[1]
        acc = jnp.zeros((x_ref.shape[0], _CHUNK), jnp.float32)
        for j in range(0, f, _CHUNK):
            blk = x_ref[:, j : j + _CHUNK].astype(jnp.float32)
            acc = acc + blk * blk
        ssq = jnp.sum(acc, axis=-1, keepdims=True)
        pen = (jnp.sqrt(ssq) - 1.0) ** 2
        if n_valid_rows < pen.shape[0]:
            row = jax.lax.broadcasted_iota(jnp.int32, pen.shape, 0)
            last = pl.num_programs(0) - 1
            pen = jnp.where((i != last) | (row < n_valid_rows), pen, 0.0)
        out_ref[...] += jnp.sum(pen) * inv_b

    return _gp_kernel


def _gradient_penalty(x):
    B, F = x.shape
    B_pad = -(-B // _TILE_B) * _TILE_B
    F_pad = -(-F // _CHUNK) * _CHUNK
    if (B_pad, F_pad) != (B, F):
        x = jnp.pad(x, ((0, B_pad - B), (0, F_pad - F)))

    n_valid_last = _TILE_B - (B_pad - B)
    loss = pl.pallas_call(
        _make_gp_kernel(n_valid_last, 1.0 / B),
        out_shape=jax.ShapeDtypeStruct((1, 1), jnp.float32),
        grid=(B_pad // _TILE_B,),
        in_specs=[pl.BlockSpec((_TILE_B, F_pad), lambda i: (i, 0))],
        out_specs=pl.BlockSpec((1, 1), lambda i: (0, 0)),
        compiler_params=pltpu.CompilerParams(
            dimension_semantics=("arbitrary",),
            vmem_limit_bytes=64 * 1024 * 1024,
        ),
    )(x)

    return loss[0, 0]


def kernel(dydx_flat):
    return _gradient_penalty(dydx_flat)
